# baseline (speedup 1.0000x reference)
import functools

import jax
import jax.numpy as jnp
from jax import lax
from jax.experimental import pallas as pl
from jax.experimental.pallas import tpu as pltpu
from jax.experimental.pallas import tpu_sc as plsc

VOCAB = 100000
N_IDX = 81920
EMBED_DIM = 64
N_FEATURES = 4
N_OUT = N_FEATURES * N_IDX

_NUM_CORES = 2
_NUM_SUBCORES = 16
_NW = _NUM_CORES * _NUM_SUBCORES
_PER_W = N_IDX // _NW
_CHUNK = 512
_CPF = _PER_W // _CHUNK

_mesh = plsc.VectorSubcoreMesh(core_axis_name="c", subcore_axis_name="s")


@functools.partial(
    pl.kernel,
    mesh=_mesh,
    out_type=jax.ShapeDtypeStruct((EMBED_DIM, N_OUT), jnp.float32),
    scratch_types=[
        pltpu.VMEM((N_FEATURES * _PER_W,), jnp.int32),
        pltpu.VMEM((EMBED_DIM, _CHUNK), jnp.float32),
        pltpu.VMEM((EMBED_DIM, _CHUNK), jnp.float32),
        pltpu.SemaphoreType.DMA,
        pltpu.SemaphoreType.DMA,
        pltpu.SemaphoreType.DMA,
        pltpu.SemaphoreType.DMA,
    ],
    compiler_params=pltpu.CompilerParams(use_tc_tiling_on_sc=False),
)
def _gather_kernel(
    idx0, idx1, idx2, idx3, t0, t1, t2, t3, out, idx_all, rA, rB, gA, gB, sA, sB
):
    wid = lax.axis_index("s") * _NUM_CORES + lax.axis_index("c")
    base = wid * _PER_W

    for f, idx_hbm in enumerate((idx0, idx1, idx2, idx3)):
        pltpu.sync_copy(
            idx_hbm.at[pl.ds(base, _PER_W)], idx_all.at[pl.ds(f * _PER_W, _PER_W)]
        )

    tabs = (t0, t1, t2, t3)
    rows = (rA, rB)
    gsems = (gA, gB)
    ssems = (sA, sB)

    def fire(c, b):
        f, k = c // _CPF, c % _CPF
        idx_v = idx_all.at[pl.ds(f * _PER_W + k * _CHUNK, _CHUNK)]
        for d in range(EMBED_DIM):
            pltpu.make_async_copy(
                tabs[f].at[d].at[idx_v],
                rows[b].at[d],
                gsems[b],
            ).start()

    def drain_gather(b):
        pltpu.make_async_copy(
            out.at[:, pl.ds(0, _CHUNK)], rows[b], gsems[b]
        ).wait()

    def store(c, b):
        f, k = c // _CPF, c % _CPF
        col = f * N_IDX + base + k * _CHUNK
        return pltpu.async_copy(rows[b], out.at[:, pl.ds(col, _CHUNK)], ssems[b])

    n_ch = N_FEATURES * _CPF
    stores = [None, None]
    fire(0, 0)
    for c in range(n_ch):
        b = c % 2
        drain_gather(b)
        stores[b] = store(c, b)
        if c + 1 < n_ch:
            nb = (c + 1) % 2
            if stores[nb] is not None:
                stores[nb].wait()
            fire(c + 1, nb)
    stores[(n_ch - 1) % 2].wait()


def kernel(idx0, idx1, idx2, idx3, table0, table1, table2, table3):
    t_ts = tuple(t.T for t in (table0, table1, table2, table3))
    out_t = _gather_kernel(idx0, idx1, idx2, idx3, *t_ts)
    return out_t.T


# speedup vs baseline: 2.2147x; 2.2147x over previous
import functools

import jax
import jax.numpy as jnp
from jax import lax
from jax.experimental import pallas as pl
from jax.experimental.pallas import tpu as pltpu
from jax.experimental.pallas import tpu_sc as plsc

N_IDX = 81920
EMBED_DIM = 64
N_FEATURES = 4

_NUM_CORES = 2
_NUM_SUBCORES = 16
_NW = _NUM_CORES * _NUM_SUBCORES
_PER_W = N_IDX // _NW
_CHUNK = 512
_N_CHUNKS = _PER_W // _CHUNK
_GROUPS = _CHUNK // 16

_mesh = plsc.VectorSubcoreMesh(core_axis_name="c", subcore_axis_name="s")


@functools.partial(
    pl.kernel,
    mesh=_mesh,
    out_type=jax.ShapeDtypeStruct((N_IDX, EMBED_DIM), jnp.float32),
    scratch_types=[
        pltpu.VMEM((_PER_W,), jnp.int32),
        pltpu.VMEM((_CHUNK, EMBED_DIM), jnp.float32),
        pltpu.SemaphoreType.DMA,
    ],
)
def _feature_gather(idx_hbm, tab, out, idx_all, rows, gsem):
    wid = lax.axis_index("s") * _NUM_CORES + lax.axis_index("c")
    base = wid * _PER_W

    pltpu.sync_copy(idx_hbm.at[pl.ds(base, _PER_W)], idx_all)

    def chunk_body(c, _):
        def group_body(g, _):
            idx_v = idx_all[pl.ds(c * _CHUNK + g * 16, 16)]
            rs = [
                jax.lax.squeeze(jax.lax.slice(idx_v, (j,), (j + 1,)), (0,))
                for j in range(16)
            ]
            for j, r in enumerate(rs):
                pltpu.make_async_copy(
                    tab.at[pl.ds(r, 1)], rows.at[pl.ds(g * 16 + j, 1)], gsem
                ).start()
            return 0

        lax.fori_loop(0, _GROUPS, group_body, 0)
        pltpu.make_async_copy(tab.at[pl.ds(0, _CHUNK)], rows, gsem).wait()
        pltpu.sync_copy(rows, out.at[pl.ds(base + c * _CHUNK, _CHUNK)])
        return 0

    lax.fori_loop(0, _N_CHUNKS, chunk_body, 0)


def kernel(idx0, idx1, idx2, idx3, table0, table1, table2, table3):
    outs = [
        _feature_gather(idx, tab)
        for idx, tab in (
            (idx0, table0),
            (idx1, table1),
            (idx2, table2),
            (idx3, table3),
        )
    ]
    return jnp.concatenate(outs, axis=0)
